# baseline (speedup 1.0000x reference)
import functools

import jax
import jax.numpy as jnp
from jax import lax
from jax.experimental import pallas as pl
from jax.experimental.pallas import tpu as pltpu
from jax.experimental.pallas import tpu_sc as plsc

EMBED = 64
NCHUNK = 4


def kernel(next_id, next_category, next_subcategory, id_table, category_table,
           subcategory_table):
    B = next_id.shape[0]
    next_id = next_id.astype(jnp.int32)
    next_category = next_category.astype(jnp.int32)
    next_subcategory = next_subcategory.astype(jnp.int32)
    cat_rows = category_table.shape[0]
    sub_rows = subcategory_table.shape[0]

    info = plsc.get_sparse_core_info()
    nw = info.num_cores * info.num_subcores
    b_per_w = B // nw
    chunk = b_per_w // NCHUNK
    ngroups = chunk // 16

    mesh = plsc.VectorSubcoreMesh(core_axis_name="c", subcore_axis_name="s")

    @functools.partial(
        pl.kernel,
        mesh=mesh,
        out_type=jax.ShapeDtypeStruct((B, 3 * EMBED), jnp.float32),
        compiler_params=pltpu.CompilerParams(use_tc_tiling_on_sc=False,
                                             needs_layout_passes=False),
        scratch_types=[
            pltpu.VMEM((b_per_w,), jnp.int32),
            pltpu.VMEM((b_per_w,), jnp.int32),
            pltpu.VMEM((b_per_w,), jnp.int32),
            pltpu.VMEM((cat_rows, EMBED), jnp.float32),
            pltpu.VMEM((sub_rows, EMBED), jnp.float32),
            [pltpu.VMEM((chunk, EMBED), jnp.float32) for _ in range(2)],
            [pltpu.VMEM((chunk, 3 * EMBED), jnp.float32) for _ in range(2)],
            [pltpu.SemaphoreType.DMA for _ in range(2)],
            [pltpu.SemaphoreType.DMA for _ in range(2)],
            pltpu.SemaphoreType.DMA,
        ],
    )
    def gather_concat(id_idx_hbm, cat_idx_hbm, sub_idx_hbm, id_tab, cat_tab,
                      sub_tab, out_hbm, idx0, idx1, idx2, cat_v, sub_v,
                      rows_id, staged, gsem, wsem, tsem):
        wid = lax.axis_index("s") * info.num_cores + lax.axis_index("c")
        base = wid * b_per_w
        tcopy0 = pltpu.async_copy(cat_tab, cat_v, tsem)
        tcopy1 = pltpu.async_copy(sub_tab, sub_v, tsem)
        pltpu.sync_copy(id_idx_hbm.at[pl.ds(base, b_per_w)], idx0)
        pltpu.sync_copy(cat_idx_hbm.at[pl.ds(base, b_per_w)], idx1)
        pltpu.sync_copy(sub_idx_hbm.at[pl.ds(base, b_per_w)], idx2)

        def fire_gather(c):
            return pltpu.async_copy(
                id_tab.at[idx0.at[pl.ds(c * chunk, chunk)]],
                rows_id[c % 2], gsem[c % 2])

        iota = jax.lax.iota(jnp.int32, 16)

        gathers = [fire_gather(0), None]
        writes = [None, None]
        tcopy0.wait()
        tcopy1.wait()
        for c in range(NCHUNK):
            p = c % 2
            if c + 1 < NCHUNK:
                gathers[1 - p] = fire_gather(c + 1)
            if writes[p] is not None:
                writes[p].wait()
                writes[p] = None
            st = staged[p]

            def cs_body(g, _, c=c, st=st):
                i16 = g * 16 + iota
                off = c * chunk + g * 16
                idxc = idx1[pl.ds(off, 16)]
                idxs = idx2[pl.ds(off, 16)]
                for col in range(EMBED):
                    colv = jnp.full((16,), col, jnp.int32)
                    v = plsc.load_gather(cat_v, [idxc, colv])
                    plsc.store_scatter(st, [i16, colv + EMBED], v)
                    w = plsc.load_gather(sub_v, [idxs, colv])
                    plsc.store_scatter(st, [i16, colv + 2 * EMBED], w)
                return 0

            lax.fori_loop(0, ngroups, cs_body, 0)
            gathers[p].wait()
            rid = rows_id[p]

            def id_body(r, _, st=st, rid=rid):
                for k in range(EMBED // 16):
                    st[r, pl.ds(16 * k, 16)] = rid[r, pl.ds(16 * k, 16)]
                return 0

            lax.fori_loop(0, chunk, id_body, 0)
            writes[p] = pltpu.async_copy(
                st, out_hbm.at[pl.ds(base + c * chunk, chunk)], wsem[p])
        for w in writes:
            if w is not None:
                w.wait()

    return gather_concat(next_id, next_category, next_subcategory, id_table,
                         category_table, subcategory_table)


# speedup vs baseline: 1.2656x; 1.2656x over previous
import functools

import jax
import jax.numpy as jnp
from jax import lax
from jax.experimental import pallas as pl
from jax.experimental.pallas import tpu as pltpu
from jax.experimental.pallas import tpu_sc as plsc

EMBED = 64


def kernel(next_id, next_category, next_subcategory, id_table, category_table,
           subcategory_table):
    B = next_id.shape[0]
    next_id = next_id.astype(jnp.int32)
    next_category = next_category.astype(jnp.int32)
    next_subcategory = next_subcategory.astype(jnp.int32)

    info = plsc.get_sparse_core_info()
    nw = info.num_cores * info.num_subcores
    b_per_w = B // nw

    mesh = plsc.VectorSubcoreMesh(core_axis_name="c", subcore_axis_name="s")

    @functools.partial(
        pl.kernel,
        mesh=mesh,
        out_type=jax.ShapeDtypeStruct((B, 3 * EMBED), jnp.float32),
        compiler_params=pltpu.CompilerParams(use_tc_tiling_on_sc=False),
        scratch_types=[
            pltpu.VMEM((b_per_w,), jnp.int32),
            pltpu.VMEM((b_per_w,), jnp.int32),
            pltpu.VMEM((b_per_w,), jnp.int32),
            pltpu.VMEM((b_per_w, EMBED), jnp.float32),
            pltpu.VMEM((b_per_w, EMBED), jnp.float32),
            pltpu.VMEM((b_per_w, EMBED), jnp.float32),
            [pltpu.SemaphoreType.DMA for _ in range(3)],
            [pltpu.SemaphoreType.DMA for _ in range(3)],
            pltpu.SemaphoreType.DMA,
        ],
    )
    def gather_concat(id_idx_hbm, cat_idx_hbm, sub_idx_hbm, id_tab, cat_tab,
                      sub_tab, out_hbm, idx0, idx1, idx2, rows0, rows1, rows2,
                      gsem, wsem, isem):
        wid = lax.axis_index("s") * info.num_cores + lax.axis_index("c")
        base = wid * b_per_w
        with jax.named_scope("idx_load"):
            i0 = pltpu.async_copy(id_idx_hbm.at[pl.ds(base, b_per_w)], idx0, isem)
            i1 = pltpu.async_copy(cat_idx_hbm.at[pl.ds(base, b_per_w)], idx1, isem)
            i2 = pltpu.async_copy(sub_idx_hbm.at[pl.ds(base, b_per_w)], idx2, isem)
            i0.wait(); i1.wait(); i2.wait()
        rows = (rows0, rows1, rows2)
        tabs = (id_tab, cat_tab, sub_tab)
        idxs = (idx0, idx1, idx2)
        with jax.named_scope("gather_fire"):
            gathers = [pltpu.async_copy(tabs[t].at[idxs[t]], rows[t], gsem[t])
                       for t in range(3)]
        writes = []
        for t in range(3):
            with jax.named_scope(f"gather_wait{t}"):
                gathers[t].wait()
            with jax.named_scope(f"write_fire{t}"):
                writes.append(pltpu.async_copy(
                    rows[t],
                    out_hbm.at[pl.ds(base, b_per_w), pl.ds(t * EMBED, EMBED)],
                    wsem[t]))
        for t in range(3):
            with jax.named_scope(f"write_wait{t}"):
                writes[t].wait()

    return gather_concat(next_id, next_category, next_subcategory, id_table,
                         category_table, subcategory_table)


# speedup vs baseline: 1.2757x; 1.0080x over previous
import functools

import jax
import jax.numpy as jnp
from jax import lax
from jax.experimental import pallas as pl
from jax.experimental.pallas import tpu as pltpu
from jax.experimental.pallas import tpu_sc as plsc

EMBED = 64
NSPLIT = 4


def kernel(next_id, next_category, next_subcategory, id_table, category_table,
           subcategory_table):
    B = next_id.shape[0]
    next_id = next_id.astype(jnp.int32)
    next_category = next_category.astype(jnp.int32)
    next_subcategory = next_subcategory.astype(jnp.int32)

    info = plsc.get_sparse_core_info()
    nw = info.num_cores * info.num_subcores
    b_per_w = B // nw
    piece = b_per_w // NSPLIT

    mesh = plsc.VectorSubcoreMesh(core_axis_name="c", subcore_axis_name="s")

    @functools.partial(
        pl.kernel,
        mesh=mesh,
        out_type=jax.ShapeDtypeStruct((B, 3 * EMBED), jnp.float32),
        compiler_params=pltpu.CompilerParams(use_tc_tiling_on_sc=False),
        scratch_types=[
            pltpu.VMEM((b_per_w,), jnp.int32),
            pltpu.VMEM((b_per_w,), jnp.int32),
            pltpu.VMEM((b_per_w,), jnp.int32),
            pltpu.VMEM((b_per_w, EMBED), jnp.float32),
            pltpu.VMEM((b_per_w, EMBED), jnp.float32),
            pltpu.VMEM((b_per_w, EMBED), jnp.float32),
            [pltpu.SemaphoreType.DMA for _ in range(NSPLIT)],
            [pltpu.SemaphoreType.DMA for _ in range(2)],
            [pltpu.SemaphoreType.DMA for _ in range(3)],
            pltpu.SemaphoreType.DMA,
        ],
    )
    def gather_concat(id_idx_hbm, cat_idx_hbm, sub_idx_hbm, id_tab, cat_tab,
                      sub_tab, out_hbm, idx0, idx1, idx2, rows0, rows1, rows2,
                      gsem0, gsem12, wsem, isem):
        wid = lax.axis_index("s") * info.num_cores + lax.axis_index("c")
        base = wid * b_per_w
        i0 = pltpu.async_copy(id_idx_hbm.at[pl.ds(base, b_per_w)], idx0, isem)
        i1 = pltpu.async_copy(cat_idx_hbm.at[pl.ds(base, b_per_w)], idx1, isem)
        i2 = pltpu.async_copy(sub_idx_hbm.at[pl.ds(base, b_per_w)], idx2, isem)
        i0.wait(); i1.wait(); i2.wait()
        id_gathers = [
            pltpu.async_copy(
                id_tab.at[idx0.at[pl.ds(k * piece, piece)]],
                rows0.at[pl.ds(k * piece, piece)], gsem0[k])
            for k in range(NSPLIT)
        ]
        g1 = pltpu.async_copy(cat_tab.at[idx1], rows1, gsem12[0])
        g2 = pltpu.async_copy(sub_tab.at[idx2], rows2, gsem12[1])
        g1.wait()
        w1 = pltpu.async_copy(
            rows1, out_hbm.at[pl.ds(base, b_per_w), pl.ds(EMBED, EMBED)],
            wsem[1])
        g2.wait()
        w2 = pltpu.async_copy(
            rows2, out_hbm.at[pl.ds(base, b_per_w), pl.ds(2 * EMBED, EMBED)],
            wsem[2])
        for g in id_gathers:
            g.wait()
        w0 = pltpu.async_copy(
            rows0, out_hbm.at[pl.ds(base, b_per_w), pl.ds(0, EMBED)], wsem[0])
        w1.wait()
        w2.wait()
        w0.wait()

    return gather_concat(next_id, next_category, next_subcategory, id_table,
                         category_table, subcategory_table)
